# baseline (speedup 1.0000x reference)
import functools

import jax
import jax.numpy as jnp
from jax import lax
from jax.experimental import pallas as pl
from jax.experimental.pallas import tpu as pltpu
from jax.experimental.pallas import tpu_sc as plsc

_Q = 0.7
_K = 0.5
_B = 16384
_NCLS = 1000
_NCORES = 2
_NSUB = 16
_NW = _NCORES * _NSUB
_PER_W = _B // _NW
_CHUNK = 128
_NCH = _PER_W // _CHUNK
_LANES = 16
_CONST = (1.0 - _K ** _Q) / _Q


def _sc_gather(logits_flat, weight_flat, targets, indexes):
    mesh = plsc.VectorSubcoreMesh(core_axis_name="c", subcore_axis_name="s")

    @functools.partial(
        pl.kernel,
        mesh=mesh,
        out_type=(
            jax.ShapeDtypeStruct((_B,), jnp.float32),
            jax.ShapeDtypeStruct((_B,), jnp.float32),
        ),
        scratch_types=[
            pltpu.VMEM((_PER_W,), jnp.int32),
            pltpu.VMEM((_PER_W,), jnp.int32),
            pltpu.VMEM((_PER_W,), jnp.int32),
            pltpu.VMEM((_PER_W,), jnp.float32),
            pltpu.VMEM((_PER_W,), jnp.float32),
            pltpu.SemaphoreType.DMA,
        ],
    )
    def gather_kernel(logits_hbm, weight_hbm, targets_hbm, indexes_hbm,
                      yg_out, w_out, tgt_v, widx_v, lidx_v, yg_v, w_v, sem):
        wid = lax.axis_index("c") * _NSUB + lax.axis_index("s")
        base = wid * _PER_W
        pltpu.sync_copy(targets_hbm.at[pl.ds(base, _PER_W)], tgt_v)
        pltpu.sync_copy(indexes_hbm.at[pl.ds(base, _PER_W)], widx_v)
        for j in range(_PER_W // _LANES):
            row = base + j * _LANES + lax.iota(jnp.int32, _LANES)
            lidx_v[pl.ds(j * _LANES, _LANES)] = (
                tgt_v[pl.ds(j * _LANES, _LANES)] + row * _NCLS)
        copies = []
        for c in range(_NCH):
            sl = pl.ds(c * _CHUNK, _CHUNK)
            copies.append(pltpu.async_copy(
                logits_hbm.at[lidx_v.at[sl]], yg_v.at[sl], sem))
            copies.append(pltpu.async_copy(
                weight_hbm.at[widx_v.at[sl]], w_v.at[sl], sem))
        for cp in copies:
            cp.wait()
        pltpu.sync_copy(yg_v, yg_out.at[pl.ds(base, _PER_W)])
        pltpu.sync_copy(w_v, w_out.at[pl.ds(base, _PER_W)])

    return gather_kernel(logits_flat, weight_flat, targets, indexes)


def _loss_body(yg_ref, w_ref, out_ref):
    yg = yg_ref[...]
    w = w_ref[...]
    p = jnp.exp(jnp.log(yg) * _Q)
    loss = ((1.0 - p) * (1.0 / _Q) - _CONST) * w
    out_ref[0, 0] = jnp.sum(loss) * (1.0 / _B)


def kernel(logits, targets, indexes, weight):
    logits_flat = logits.reshape(_B * _NCLS)
    weight_flat = weight.reshape(-1)
    tgt = targets.astype(jnp.int32)
    idx = indexes.astype(jnp.int32)
    yg, w = _sc_gather(logits_flat, weight_flat, tgt, idx)
    out = pl.pallas_call(
        _loss_body,
        out_shape=jax.ShapeDtypeStruct((1, 1), jnp.float32),
        out_specs=pl.BlockSpec(memory_space=pltpu.SMEM),
    )(yg.reshape(128, 128), w.reshape(128, 128))
    return out[0, 0]
